# baseline (speedup 1.0000x reference)
import functools

import jax
import jax.numpy as jnp
from jax import lax
from jax.experimental import pallas as pl
from jax.experimental.pallas import tpu as pltpu
from jax.experimental.pallas import tpu_sc as plsc


def kernel(tokens, W_pos):
    batch, seq_len = tokens.shape
    d_model = W_pos.shape[1]

    info = plsc.get_sparse_core_info()
    NC, NS = info.num_cores, info.num_subcores
    NW = NC * NS
    rows_per_w = seq_len // NW
    CA = 16
    CB = 24
    NA = 5
    NB = 2
    assert NA * CA + NB * CB == rows_per_w

    mesh = plsc.VectorSubcoreMesh(core_axis_name="c", subcore_axis_name="s")

    @functools.partial(
        pl.kernel,
        mesh=mesh,
        out_type=jax.ShapeDtypeStruct((batch, seq_len, d_model), W_pos.dtype),
        scratch_types=[
            pltpu.VMEM((CA, d_model), jnp.float32),
            pltpu.VMEM_SHARED((NS, NB, CB, d_model), jnp.float32),
            pltpu.SemaphoreType.DMA,
            pltpu.SemaphoreType.DMA,
            pltpu.SemaphoreType.DMA,
            pltpu.SemaphoreType.DMA,
        ],
    )
    def sc_broadcast(w_hbm, out_hbm, buf, sbuf, rsem0, rsem1, wsem0, wsem1):
        cid = lax.axis_index("c")
        sid = lax.axis_index("s")
        wid = sid * NC + cid
        base0 = wid * rows_per_w
        baseB = [base0 + NA * CA + j * CB for j in range(NB)]
        rsems = (rsem0, rsem1)
        wsems = (wsem0, wsem1)
        hbr = [
            pltpu.async_copy(
                w_hbm.at[pl.ds(baseB[j], CB)], sbuf.at[sid, j], rsems[j])
            for j in range(NB)
        ]
        hbw = [None] * NB
        for c in range(NA):
            base = base0 + c * CA
            pltpu.sync_copy(w_hbm.at[pl.ds(base, CA)], buf)
            for b in range(batch):
                pltpu.sync_copy(buf, out_hbm.at[b, pl.ds(base, CA)])
            if c < NB:
                hbr[c].wait()
                hbw[c] = [
                    pltpu.async_copy(
                        sbuf.at[sid, c], out_hbm.at[b, pl.ds(baseB[c], CB)],
                        wsems[c])
                    for b in range(batch)
                ]
        for j in range(NB):
            for h in hbw[j]:
                h.wait()

    return sc_broadcast(W_pos)


# speedup vs baseline: 1.0119x; 1.0119x over previous
import functools

import jax
import jax.numpy as jnp
from jax import lax
from jax.experimental import pallas as pl
from jax.experimental.pallas import tpu as pltpu
from jax.experimental.pallas import tpu_sc as plsc


def kernel(tokens, W_pos):
    batch, seq_len = tokens.shape
    d_model = W_pos.shape[1]

    info = plsc.get_sparse_core_info()
    NC, NS = info.num_cores, info.num_subcores
    NW = NC * NS
    rows_per_w = seq_len // NW
    C = 32
    n_chunks = rows_per_w // C

    mesh = plsc.VectorSubcoreMesh(core_axis_name="c", subcore_axis_name="s")

    @functools.partial(
        pl.kernel,
        mesh=mesh,
        out_type=jax.ShapeDtypeStruct((batch, seq_len, d_model), W_pos.dtype),
        scratch_types=[
            pltpu.VMEM((C, d_model), jnp.float32),
            pltpu.VMEM_SHARED((NS, C, d_model), jnp.float32),
            pltpu.SemaphoreType.DMA,
            pltpu.SemaphoreType.DMA,
        ],
    )
    def sc_broadcast(w_hbm, out_hbm, buf, sbuf, rsem, wsem):
        cid = lax.axis_index("c")
        sid = lax.axis_index("s")
        wid = sid * NC + cid
        base0 = wid * rows_per_w
        baseB = base0 + (n_chunks - 1) * C
        hb_read = pltpu.async_copy(w_hbm.at[pl.ds(baseB, C)], sbuf.at[sid], rsem)
        hb_writes = None
        for c in range(n_chunks - 1):
            base = base0 + c * C
            pltpu.sync_copy(w_hbm.at[pl.ds(base, C)], buf)
            for b in range(batch):
                pltpu.sync_copy(buf, out_hbm.at[b, pl.ds(base, C)])
            if c == 0:
                hb_read.wait()
                hb_writes = [
                    pltpu.async_copy(
                        sbuf.at[sid], out_hbm.at[b, pl.ds(baseB, C)], wsem)
                    for b in range(batch)
                ]
        for h in hb_writes:
            h.wait()

    return sc_broadcast(W_pos)
